# Initial kernel scaffold; baseline (speedup 1.0000x reference)
import jax
import jax.numpy as jnp
from jax.experimental import pallas as pl


def kernel(collabs, contexts, cW1, cb1, cW21, cb21, cW22, cb22, cW3, cb3, cW4, cb4, xW1, xb1, xW21, xb21, xW22, xb22, xW3, xb3, xW4, xb4):
    raise NotImplementedError("write your pallas kernel here")


# speedup vs baseline: 1.2602x; 1.2602x over previous
import functools
import math

import numpy as np
import jax
import jax.numpy as jnp
from jax import lax
from jax.experimental import pallas as pl
from jax.experimental.pallas import tpu as pltpu
from jax.experimental.pallas import tpu_sc as plsc

STD2 = 0.1
VAR2 = STD2 * STD2
LOG_VAR2 = math.log(VAR2)

N_COLLAB = 4096
N_CTX = 16384
D_IN = 384
D_LAT = 64

BLK = 256
VBLK = 1024

_PERM = np.arange(4096, dtype=np.int32)
_PERM = (_PERM % 32) * 128 + _PERM // 32


def _dot_t(x, w):
    return lax.dot_general(x.astype(jnp.bfloat16), w.astype(jnp.bfloat16),
                           (((1,), (1,)), ((), ())),
                           preferred_element_type=jnp.float32)


def _vae_latents(x, eps, w1, b1, w21, b21, w22, b22):
    h = jnp.maximum(_dot_t(x, w1) + b1, 0.0)
    mu = _dot_t(h, w21) + b21
    log_var = _dot_t(h, w22) + b22
    std = jnp.exp(0.5 * log_var) * STD2
    z = mu + eps * std
    return z, mu, log_var


def _vae_decode(z, w3, b3, w4, b4):
    h2 = jnp.maximum(_dot_t(z, w3) + b3, 0.0)
    return _dot_t(h2, w4) + b4


def _loss_sums(x_hat, x, mu, log_var):
    mse_sum = jnp.sum((x_hat - x) ** 2)
    kld_sum = jnp.sum(1.0 - LOG_VAR2 + log_var
                      - (mu ** 2 + jnp.exp(log_var)) / VAR2)
    return mse_sum, kld_sum


def _sq_norm64(z):
    n = z.shape[0]
    t = (z * z).reshape(n, 8, 8)
    a = t[:, 0, :]
    for i in range(1, 8):
        a = a + t[:, i, :]
    while a.shape[1] > 1:
        h = a.shape[1] // 2
        a = a[:, :h] + a[:, h:]
    return a


def _l2norm(z):
    n = jnp.sqrt(_sq_norm64(z))
    return z / jnp.maximum(n, 1e-12)


def _vae_kernel(x_ref, eps_ref, w1, b1, w21, b21, w22, b22, w3, b3,
                w4, b4, zn_ref, mse_ref, kld_ref):
    @pl.when(pl.program_id(0) == 0)
    def _init():
        mse_ref[...] = jnp.zeros((1, 1), jnp.float32)
        kld_ref[...] = jnp.zeros((1, 1), jnp.float32)

    x = x_ref[...]
    z, mu, log_var = _vae_latents(x, eps_ref[...], w1[...], b1[...],
                                  w21[...], b21[...], w22[...], b22[...])
    x_hat = _vae_decode(z, w3[...], b3[...], w4[...], b4[...])
    mse_sum, kld_sum = _loss_sums(x_hat, x, mu, log_var)
    zn_ref[...] = _l2norm(z)
    mse_ref[...] += jnp.reshape(mse_sum, (1, 1))
    kld_ref[...] += jnp.reshape(kld_sum, (1, 1))


def _run_vae(x, eps, weights, n_rows):
    full = lambda a: pl.BlockSpec(a.shape, lambda *_: (0,) * a.ndim)
    return pl.pallas_call(
        _vae_kernel,
        grid=(n_rows // VBLK,),
        in_specs=[
            pl.BlockSpec((VBLK, D_IN), lambda i: (i, 0)),
            pl.BlockSpec((VBLK, D_LAT), lambda i: (i, 0)),
            *[full(w) for w in weights],
        ],
        out_specs=[
            pl.BlockSpec((VBLK, D_LAT), lambda i: (i, 0)),
            pl.BlockSpec((1, 1), lambda i: (0, 0)),
            pl.BlockSpec((1, 1), lambda i: (0, 0)),
        ],
        out_shape=(
            jax.ShapeDtypeStruct((n_rows, D_LAT), jnp.float32),
            jax.ShapeDtypeStruct((1, 1), jnp.float32),
            jax.ShapeDtypeStruct((1, 1), jnp.float32),
        ),
    )(x, eps, *weights)


def _sample_kernel(p_ref, rand_ref, sel_ref, logp_ref):
    p_sm = p_ref[...].reshape(128, 32, BLK)
    r = rand_ref[...]
    slabs = [p_sm[0]]
    for si in range(1, 128):
        slabs.append(slabs[-1] + p_sm[si])
    tot = slabs[-1]
    offs = [jnp.zeros((1, BLK), jnp.float32)]
    run = tot[0:1]
    for b in range(1, 32):
        offs.append(run)
        run = run + tot[b:b + 1]
    off = jnp.concatenate(offs, axis=0)
    cnt = jnp.zeros((32, BLK), jnp.int32)
    for si in range(128):
        cnt += ((slabs[si] + off) <= r).astype(jnp.int32)
    cnt = jnp.sum(cnt, axis=0, keepdims=True)
    sel = jnp.where(cnt >= N_COLLAB, 0, cnt)
    sel_ref[...] = sel.reshape(1, 1, BLK)

    b_iota = lax.broadcasted_iota(jnp.int32, (32, BLK), 0)
    p_hit = jnp.zeros((32, BLK), jnp.float32)
    for si in range(128):
        idx = b_iota * 128 + si
        p_hit += jnp.where(idx == sel, p_sm[si], 0.0)
    p_sel = jnp.sum(p_hit, axis=0, keepdims=True)
    logp_ref[...] = jnp.log(p_sel).reshape(1, 1, BLK)


GATHER_W = 128


def _make_sc_gather():
    info = plsc.get_sparse_core_info()
    nw = info.num_cores * info.num_subcores
    bpw = N_CTX // nw
    mesh = plsc.VectorSubcoreMesh(core_axis_name="c", subcore_axis_name="s")

    @functools.partial(
        pl.kernel, mesh=mesh,
        out_type=jax.ShapeDtypeStruct((N_CTX, GATHER_W), jnp.float32),
        scratch_types=[
            pltpu.VMEM((bpw,), jnp.int32),
            pltpu.VMEM((bpw, GATHER_W), jnp.float32),
            pltpu.SemaphoreType.DMA,
        ],
    )
    def gather(table_hbm, idx_hbm, out_hbm, idx_v, rows_v, sem):
        wid = lax.axis_index("s") * info.num_cores + lax.axis_index("c")
        base = wid * bpw
        pltpu.sync_copy(idx_hbm.at[pl.ds(base, bpw)], idx_v)
        pltpu.async_copy(table_hbm.at[idx_v], rows_v, sem).wait()
        pltpu.sync_copy(rows_v, out_hbm.at[pl.ds(base, bpw)])

    return gather


def _gather_rows(table, idx):
    padded = jnp.pad(table, ((0, 0), (0, GATHER_W - table.shape[1])))
    return _make_sc_gather()(padded, idx)[:, :table.shape[1]]


def kernel(collabs, contexts, cW1, cb1, cW21, cb21, cW22, cb22, cW3, cb3,
           cW4, cb4, xW1, xb1, xW21, xb21, xW22, xb22, xW3, xb3, xW4, xb4):
    temp = 1.0
    eps_c = jax.random.normal(jax.random.key(1), (N_COLLAB, D_LAT), jnp.float32)
    eps_x = jax.random.normal(jax.random.key(2), (N_CTX, D_LAT), jnp.float32)
    rand_t = jax.random.uniform(jax.random.key(3), (N_CTX, 1),
                                jnp.float32).reshape(1, N_CTX)

    cw = [cW1, cb1.reshape(1, -1), cW21, cb21.reshape(1, -1),
          cW22, cb22.reshape(1, -1), cW3, cb3.reshape(1, -1),
          cW4, cb4.reshape(1, -1)]
    xw = [xW1, xb1.reshape(1, -1), xW21, xb21.reshape(1, -1),
          xW22, xb22.reshape(1, -1), xW3, xb3.reshape(1, -1),
          xW4, xb4.reshape(1, -1)]

    collab_zn, c_mse, c_kld = _run_vae(collabs, eps_c, cw, N_COLLAB)
    ctx_zn, x_mse, x_kld = _run_vae(contexts, eps_x, xw, N_CTX)

    scores = ctx_zn @ collab_zn.T
    p = jax.nn.softmax(scores / temp, axis=1)
    p_sm = p.T[_PERM]

    grid = (N_CTX // BLK,)
    sel, logp = pl.pallas_call(
        _sample_kernel,
        grid=grid,
        in_specs=[
            pl.BlockSpec((N_COLLAB, BLK), lambda i: (0, i)),
            pl.BlockSpec((1, BLK), lambda i: (0, i)),
        ],
        out_specs=[
            pl.BlockSpec((1, 1, BLK), lambda i: (i, 0, 0)),
            pl.BlockSpec((1, 1, BLK), lambda i: (i, 0, 0)),
        ],
        out_shape=(
            jax.ShapeDtypeStruct((N_CTX // BLK, 1, BLK), jnp.int32),
            jax.ShapeDtypeStruct((N_CTX // BLK, 1, BLK), jnp.float32),
        ),
    )(p_sm, rand_t)

    selected_index = sel.reshape(N_CTX)
    log_probs = logp.reshape(N_CTX, 1)
    collab_embedding = _gather_rows(collab_zn, selected_index)

    loss_c = c_mse[0, 0] / (N_COLLAB * D_IN) + \
        (-0.5) * c_kld[0, 0] / (N_COLLAB * D_LAT)
    loss_x = x_mse[0, 0] / (N_CTX * D_IN) + \
        (-0.5) * x_kld[0, 0] / (N_CTX * D_LAT)
    vae_loss = loss_c + loss_x

    return (selected_index, log_probs, collab_embedding, vae_loss)
